# baseline (speedup 1.0000x reference)
import functools

import jax
import jax.numpy as jnp
from jax import lax
from jax.experimental import pallas as pl
from jax.experimental.pallas import tpu as pltpu
from jax.experimental.pallas import tpu_sc as plsc

NV = 26
DEPTH = 32
DE = 7
DM = 128
B = 1024
NTOK = NV * B * DEPTH
_LR = 8
NROWS = _LR * NV

_NC = 2
_NS = 16
_NW = _NC * _NS
_RPW = NV * B // _NW
_S = DEPTH
_CH = 32
_NCH = _RPW // _CH
_NB = 8


def _lut_body(emb_ref, w_ref, b_ref, g_ref, bt_ref, lut_ref):
    rowmask = (lax.broadcasted_iota(jnp.int32, (_LR, 1), 0) < 2).astype(jnp.float32)
    for i in range(NV):
        e = jnp.concatenate(
            [emb_ref[i], jnp.zeros((_LR - 3, DE), jnp.float32)]) * rowmask
        h = lax.dot_general(e, w_ref[i], (((1,), (0,)), ((), ())),
                            preferred_element_type=jnp.float32)
        h = h + b_ref[i]
        mu = jnp.mean(h, axis=-1, keepdims=True)
        var = jnp.mean((h - mu) ** 2, axis=-1, keepdims=True)
        lut_ref[i] = (h - mu) * lax.rsqrt(var + 1e-5) * g_ref[i] + bt_ref[i]


def _lut(emb_tables, W, b3, g3, bt3):
    return pl.pallas_call(
        _lut_body,
        out_shape=jax.ShapeDtypeStruct((NV, _LR, DM), jnp.float32),
    )(emb_tables, W, b3, g3, bt3)


def _sc_body(lut_hbm, x_hbm, out_hbm, lut_s,
             x_v, idx_v, rows_v, sem_x, sem_g, sem_o):
    sid = lax.axis_index("s")
    wid = sid * _NC + lax.axis_index("c")
    base_r = wid * _RPW

    @pl.when(sid == 0)
    def _():
        pltpu.sync_copy(lut_hbm, lut_s)
    plsc.subcore_barrier()

    def fire_x(c, xb):
        r0 = base_r + c * _CH
        pltpu.async_copy(x_hbm.at[pl.ds(r0 % B, _CH), pl.ds(r0 // B, 1)],
                         x_v[xb], sem_x[xb])

    def wait_x(xb):
        pltpu.make_async_copy(x_hbm.at[pl.ds(0, _CH), pl.ds(0, 1)],
                              x_v[xb], sem_x[xb]).wait()

    def wait_o(u):
        pltpu.make_async_copy(rows_v[u], out_hbm.at[pl.ds(0, _S)],
                              sem_o[u]).wait()

    fire_x(0, 0)
    fire_x(1, 1)

    def half(p, c, xb):
        wait_x(xb)
        rowb = _LR * ((base_r + c * _CH) // B)
        for q in range(_CH // _NB):
            descs = []
            for u in range(_NB):
                j = q * _NB + u

                @pl.when(jnp.logical_or(p >= 1, jnp.int32(q + xb) >= 1))
                def _():
                    wait_o(u)

                for g in range(2):
                    xv = x_v[xb][j, 0, pl.ds(g * 16, 16)]
                    cc = jnp.where(xv < 255, jnp.minimum(xv, 1), 2)
                    idx_v[u][pl.ds(g * 16, 16)] = rowb + cc
                descs.append(pltpu.async_copy(lut_s.at[idx_v[u]], rows_v[u],
                                              sem_g[u]))
            for u in range(_NB):
                r0 = base_r + c * _CH + q * _NB + u
                descs[u].wait()
                pltpu.async_copy(rows_v[u], out_hbm.at[pl.ds(r0 * DEPTH, _S)],
                                 sem_o[u])

        @pl.when(p < (_NCH // 2) - 1)
        def _():
            fire_x(c + 2, xb)

    def pair(p, carry):
        half(p, 2 * p, 0)
        half(p, 2 * p + 1, 1)
        return carry

    lax.fori_loop(0, _NCH // 2, pair, 0)
    for u in range(_NB):
        wait_o(u)


@functools.cache
def _sc_expand():
    return functools.partial(
        pl.kernel,
        out_type=jax.ShapeDtypeStruct((NTOK, DM), jnp.float32),
        mesh=plsc.VectorSubcoreMesh(core_axis_name="c", subcore_axis_name="s"),
        scratch_types=[
            pltpu.VMEM_SHARED((NROWS, DM), jnp.float32),
            [pltpu.VMEM((_CH, 1, DEPTH), jnp.int32) for _ in range(2)],
            [pltpu.VMEM((_S,), jnp.int32) for _ in range(_NB)],
            [pltpu.VMEM((_S, DM), jnp.float32) for _ in range(_NB)],
            [pltpu.SemaphoreType.DMA for _ in range(2)],
            [pltpu.SemaphoreType.DMA for _ in range(_NB)],
            [pltpu.SemaphoreType.DMA for _ in range(_NB)],
        ],
    )(_sc_body)


def kernel(x, emb_tables, W, b, gamma, beta):
    lut = _lut(emb_tables, W,
               b.reshape(NV, 1, DM),
               gamma.reshape(NV, 1, DM),
               beta.reshape(NV, 1, DM))
    out = _sc_expand()(lut.reshape(NROWS, DM), x.astype(jnp.int32))
    return out.reshape(NV, B, DEPTH, DM)


# speedup vs baseline: 1.0279x; 1.0279x over previous
import functools

import jax
import jax.numpy as jnp
from jax import lax
from jax.experimental import pallas as pl
from jax.experimental.pallas import tpu as pltpu
from jax.experimental.pallas import tpu_sc as plsc

NV = 26
DEPTH = 32
DE = 7
DM = 128
B = 1024
NTOK = NV * B * DEPTH
_LR = 8
NROWS = _LR * NV

_NC = 2
_NS = 16
_NW = _NC * _NS
_RPW = NV * B // _NW
_S = DEPTH
_CH = 32
_NCH = _RPW // _CH
_NB = 16


def _lut_body(emb_ref, w_ref, b_ref, g_ref, bt_ref, lut_ref):
    rowmask = (lax.broadcasted_iota(jnp.int32, (_LR, 1), 0) < 2).astype(jnp.float32)
    for i in range(NV):
        e = jnp.concatenate(
            [emb_ref[i], jnp.zeros((_LR - 3, DE), jnp.float32)]) * rowmask
        h = lax.dot_general(e, w_ref[i], (((1,), (0,)), ((), ())),
                            preferred_element_type=jnp.float32)
        h = h + b_ref[i]
        mu = jnp.mean(h, axis=-1, keepdims=True)
        var = jnp.mean((h - mu) ** 2, axis=-1, keepdims=True)
        lut_ref[i] = (h - mu) * lax.rsqrt(var + 1e-5) * g_ref[i] + bt_ref[i]


def _lut(emb_tables, W, b3, g3, bt3):
    return pl.pallas_call(
        _lut_body,
        out_shape=jax.ShapeDtypeStruct((NV, _LR, DM), jnp.float32),
    )(emb_tables, W, b3, g3, bt3)


def _sc_body(lut_hbm, x_hbm, out_hbm, lut_s,
             x_v, idx_v, rows_v, sem_x, sem_g, sem_o):
    sid = lax.axis_index("s")
    wid = sid * _NC + lax.axis_index("c")
    base_r = wid * _RPW

    @pl.when(sid == 0)
    def _():
        pltpu.sync_copy(lut_hbm, lut_s)
    plsc.subcore_barrier()

    def fire_x(c, xb):
        r0 = base_r + c * _CH
        pltpu.async_copy(x_hbm.at[pl.ds(r0 % B, _CH), pl.ds(r0 // B, 1)],
                         x_v[xb], sem_x[xb])

    def wait_x(xb):
        pltpu.make_async_copy(x_hbm.at[pl.ds(0, _CH), pl.ds(0, 1)],
                              x_v[xb], sem_x[xb]).wait()

    def wait_o(u):
        pltpu.make_async_copy(rows_v[u], out_hbm.at[pl.ds(0, _S)],
                              sem_o[u]).wait()

    fire_x(0, 0)
    fire_x(1, 1)

    def half(p, c, xb):
        wait_x(xb)
        rowb = _LR * ((base_r + c * _CH) // B)
        for q in range(_CH // _NB):
            descs = []
            for u in range(_NB):
                j = q * _NB + u

                @pl.when(jnp.logical_or(p >= 1, jnp.int32(q + xb) >= 1))
                def _():
                    wait_o(u)

                for g in range(2):
                    xv = x_v[xb][j, 0, pl.ds(g * 16, 16)]
                    cc = jnp.where(xv < 255, jnp.minimum(xv, 1), 2)
                    idx_v[u][pl.ds(g * 16, 16)] = rowb + cc
                descs.append(pltpu.async_copy(lut_s.at[idx_v[u]], rows_v[u],
                                              sem_g))
            for u in range(_NB):
                r0 = base_r + c * _CH + q * _NB + u
                descs[u].wait()
                pltpu.async_copy(rows_v[u], out_hbm.at[pl.ds(r0 * DEPTH, _S)],
                                 sem_o[u])

        @pl.when(p < (_NCH // 2) - 1)
        def _():
            fire_x(c + 2, xb)

    def pair(p, carry):
        half(p, 2 * p, 0)
        half(p, 2 * p + 1, 1)
        return carry

    lax.fori_loop(0, _NCH // 2, pair, 0)
    for u in range(_NB):
        wait_o(u)


@functools.cache
def _sc_expand():
    return functools.partial(
        pl.kernel,
        out_type=jax.ShapeDtypeStruct((NTOK, DM), jnp.float32),
        mesh=plsc.VectorSubcoreMesh(core_axis_name="c", subcore_axis_name="s"),
        scratch_types=[
            pltpu.VMEM_SHARED((NROWS, DM), jnp.float32),
            [pltpu.VMEM((_CH, 1, DEPTH), jnp.int32) for _ in range(2)],
            [pltpu.VMEM((_S,), jnp.int32) for _ in range(_NB)],
            [pltpu.VMEM((_S, DM), jnp.float32) for _ in range(_NB)],
            [pltpu.SemaphoreType.DMA for _ in range(2)],
            pltpu.SemaphoreType.DMA,
            [pltpu.SemaphoreType.DMA for _ in range(_NB)],
        ],
    )(_sc_body)


def kernel(x, emb_tables, W, b, gamma, beta):
    lut = _lut(emb_tables, W,
               b.reshape(NV, 1, DM),
               gamma.reshape(NV, 1, DM),
               beta.reshape(NV, 1, DM))
    out = _sc_expand()(lut.reshape(NROWS, DM), x.astype(jnp.int32))
    return out.reshape(NV, B, DEPTH, DM)
